# Initial kernel scaffold; baseline (speedup 1.0000x reference)
import jax
import jax.numpy as jnp
from jax.experimental import pallas as pl


def kernel(x, W1, b1, W2, b2, Wa1, ba1, Wa2, ba2, W_ih, W_hh, b_ih, b_hh, Wo1, bo1, Wo2, bo2, Wout, bout):
    raise NotImplementedError("write your pallas kernel here")


# speedup vs baseline: 23985.4856x; 23985.4856x over previous
import jax
import jax.numpy as jnp
import numpy as np
from jax.experimental import pallas as pl

B = 8
N = 207
T = 12
FL = 12
H = 64

_TS = np.linspace(np.float32(0.0), np.float32(FL), FL, dtype=np.float32)
_DTS = tuple(float(_TS[i + 1] - _TS[i]) for i in range(FL - 1))


def _forward(x_ref, w1_ref, b1_ref, w2_ref, b2_ref, wa1_ref, ba1_ref,
             wa2_ref, ba2_ref, wir_ref, wiz_ref, win_ref, bir_ref, biz_ref,
             bin_ref, bhr_ref, bhz_ref, bhn_ref, wo1_ref, bo1_ref, wo2_ref,
             bo2_ref, wout_ref, bout_ref, out_ref):
    xm = jnp.mean(x_ref[:], axis=1)

    f1 = jnp.maximum(xm[:, :, None] * w1_ref[0][None, None, :]
                     + b1_ref[0][None, None, :], 0.0)
    f1 = f1.reshape(B * T, H)
    f2 = jnp.dot(f1, w2_ref[:], preferred_element_type=jnp.float32)
    f2 = jnp.maximum(f2 + b2_ref[0][None, :], 0.0)

    a = jnp.tanh(jnp.dot(f2, wa1_ref[:], preferred_element_type=jnp.float32)
                 + ba1_ref[0][None, :])
    e = jnp.sum(a * wa2_ref[0][None, :], axis=1) + ba2_ref[0, 0]
    e = e.reshape(B, T)
    e = e - jnp.max(e, axis=1, keepdims=True)
    w = jnp.exp(e)
    w = w / jnp.sum(w, axis=1, keepdims=True)
    nf = jnp.sum(f2.reshape(B, T, H) * w[:, :, None], axis=1)

    gr = jnp.dot(nf, wir_ref[:], preferred_element_type=jnp.float32) + bir_ref[0][None, :]
    gz = jnp.dot(nf, wiz_ref[:], preferred_element_type=jnp.float32) + biz_ref[0][None, :]
    gn = jnp.dot(nf, win_ref[:], preferred_element_type=jnp.float32) + bin_ref[0][None, :]
    r = jax.nn.sigmoid(gr + bhr_ref[0][None, :])
    z = jax.nn.sigmoid(gz + bhz_ref[0][None, :])
    ng = jnp.tanh(gn + r * bhn_ref[0][None, :])
    hidden = (1.0 - z) * ng

    def ode_f(y):
        h = jnp.tanh(jnp.dot(y, wo1_ref[:], preferred_element_type=jnp.float32)
                     + bo1_ref[0][None, :])
        return jnp.tanh(jnp.dot(h, wo2_ref[:], preferred_element_type=jnp.float32)
                        + bo2_ref[0][None, :])

    ys = [hidden]
    y = hidden
    for i in range(FL - 1):
        dt = _DTS[i]
        k1 = ode_f(y)
        k2 = ode_f(y + dt * k1 / 3.0)
        k3 = ode_f(y + dt * (k2 - k1 / 3.0))
        k4 = ode_f(y + dt * (k1 - k2 + k3))
        y = y + dt * (k1 + 3.0 * (k2 + k3) + k4) / 8.0
        ys.append(y)

    evolved = jnp.stack(ys, axis=1)
    p = jnp.sum(evolved * wout_ref[0][None, None, :], axis=2) + bout_ref[0, 0]
    out_ref[:] = jnp.broadcast_to(p[:, None, :], (B, N, FL))


def kernel(x, W1, b1, W2, b2, Wa1, ba1, Wa2, ba2, W_ih, W_hh, b_ih, b_hh,
           Wo1, bo1, Wo2, bo2, Wout, bout):
    del W_hh
    r2 = lambda v: v.reshape(1, -1)
    args = (
        x,
        W1.reshape(1, H), r2(b1),
        W2, r2(b2),
        Wa1, r2(ba1),
        Wa2.reshape(1, H), ba2.reshape(1, 1),
        W_ih[:H].T, W_ih[H:2 * H].T, W_ih[2 * H:].T,
        r2(b_ih[:H]), r2(b_ih[H:2 * H]), r2(b_ih[2 * H:]),
        r2(b_hh[:H]), r2(b_hh[H:2 * H]), r2(b_hh[2 * H:]),
        Wo1, r2(bo1),
        Wo2, r2(bo2),
        Wout.reshape(1, H), bout.reshape(1, 1),
    )
    return pl.pallas_call(
        _forward,
        out_shape=jax.ShapeDtypeStruct((B, N, FL), jnp.float32),
    )(*args)
